# baseline (speedup 1.0000x reference)
import functools

import jax
import jax.numpy as jnp
import numpy as np
from jax import lax
from jax.experimental import pallas as pl
from jax.experimental.pallas import tpu as pltpu
from jax.experimental.pallas import tpu_sc as plsc

NC = 2
NS = 16
NW = NC * NS
K = 80


def _flat_wid():
    return lax.axis_index("s") * NC + lax.axis_index("c")


def _make_deg_kernel(n_pad, nchunk):
    mesh = plsc.VectorSubcoreMesh(core_axis_name="c", subcore_axis_name="s")
    rps = n_pad // NS

    @functools.partial(
        pl.kernel,
        out_type=jax.ShapeDtypeStruct((NC, n_pad), jnp.float32),
        mesh=mesh,
        scratch_types=[
            pltpu.VMEM_SHARED((n_pad,), jnp.float32),
            pltpu.VMEM((nchunk, K), jnp.int32),
            pltpu.VMEM((K,), jnp.float32),
            pltpu.VMEM((rps,), jnp.float32),
        ],
    )
    def k(dst_hbm, out_hbm, acc, dstv, ones_v, zbuf):
        c = lax.axis_index("c")
        s = lax.axis_index("s")
        wid = _flat_wid()

        for i in range(K // 16):
            ones_v[pl.ds(16 * i, 16)] = jnp.ones((16,), jnp.float32)

        def zfill(i, _):
            zbuf[pl.ds(16 * i, 16)] = jnp.zeros((16,), jnp.float32)
            return 0
        lax.fori_loop(0, rps // 16, zfill, 0)
        pltpu.sync_copy(zbuf, acc.at[pl.ds(s * rps, rps)])

        pltpu.sync_copy(dst_hbm.at[wid], dstv)
        plsc.subcore_barrier()

        def body(ci, _):
            pltpu.sync_copy(ones_v, acc.at[dstv.at[ci]], add=True)
            return 0
        lax.fori_loop(0, nchunk, body, 0)

        plsc.subcore_barrier()
        pltpu.sync_copy(acc.at[pl.ds(s * rps, rps)],
                        out_hbm.at[c, pl.ds(s * rps, rps)])

    return k


def _make_agg_kernel(n_pad, nchunk, d, nblk):
    mesh = plsc.VectorSubcoreMesh(core_axis_name="c", subcore_axis_name="s")
    rps = n_pad // NS
    cpb = nchunk // nblk
    npairs = cpb // 2
    d32 = d // 2

    @functools.partial(
        pl.kernel,
        out_type=jax.ShapeDtypeStruct((NC, n_pad, d), jnp.float32),
        mesh=mesh,
        scratch_types=[
            pltpu.VMEM_SHARED((n_pad, d), jnp.float32),
            pltpu.VMEM((cpb, K), jnp.int32),
            pltpu.VMEM((cpb, K), jnp.int32),
            pltpu.VMEM((K, d32), jnp.uint32),
            pltpu.VMEM((K, d32), jnp.uint32),
            pltpu.VMEM((K, d), jnp.float32),
            pltpu.VMEM((K, d), jnp.float32),
            pltpu.SemaphoreType.DMA,
            pltpu.SemaphoreType.DMA,
            pltpu.SemaphoreType.DMA,
            pltpu.SemaphoreType.DMA,
        ],
        compiler_params=pltpu.CompilerParams(use_tc_tiling_on_sc=False,
                                             needs_layout_passes=False),
    )
    def k(y_hbm, src_hbm, dst_hbm, out_hbm, acc, srcv, dstv, pk0, pk1,
          f0, f1, gsem0, gsem1, ssem0, ssem1):
        c = lax.axis_index("c")
        s = lax.axis_index("s")
        wid = _flat_wid()
        ibufs = ((pk0, gsem0), (pk1, gsem1))
        fbufs = ((f0, ssem0), (f1, ssem1))

        def zfill(i, _):
            for j in range(d // 16):
                f0[i, pl.ds(16 * j, 16)] = jnp.zeros((16,), jnp.float32)
            return 0
        lax.fori_loop(0, K, zfill, 0)
        for t in range(rps // K):
            pltpu.sync_copy(f0, acc.at[pl.ds(s * rps + t * K, K), :])
        plsc.subcore_barrier()

        mask = jnp.uint32(0xFFFF0000)

        def convert(ib, fb):
            @plsc.parallel_loop(0, K, unroll=8)
            def crow(r):
                for q in range(d32 // 16):
                    u = ib[r, pl.ds(16 * q, 16)]
                    fb[r, pl.ds(32 * q, 16)] = plsc.bitcast(
                        u << jnp.uint32(16), jnp.float32)
                    fb[r, pl.ds(32 * q + 16, 16)] = plsc.bitcast(
                        u & mask, jnp.float32)

        def step(ci, b, wait_s, issue_g):
            ib, gs = ibufs[b]
            fb, ss = fbufs[b]
            pltpu.make_async_copy(y_hbm.at[srcv.at[ci]], ib, gs).wait()
            if wait_s:
                pltpu.make_async_copy(fb, acc.at[dstv.at[ci]], ss).wait()
            convert(ib, fb)
            pltpu.async_copy(fb, acc.at[dstv.at[ci]], ss, add=True)
            if issue_g:
                pltpu.async_copy(y_hbm.at[srcv.at[ci + 2]], ib, gs)

        for blk in range(nblk):
            pltpu.sync_copy(src_hbm.at[wid, pl.ds(blk * cpb, cpb), :], srcv)
            pltpu.sync_copy(dst_hbm.at[wid, pl.ds(blk * cpb, cpb), :], dstv)
            pltpu.async_copy(y_hbm.at[srcv.at[0]], pk0, gsem0)
            pltpu.async_copy(y_hbm.at[srcv.at[1]], pk1, gsem1)

            for b in range(2):
                step(b, b, False, True)

            def body(g, _):
                for b in range(2):
                    step(2 * g + b, b, True, True)
                return 0
            lax.fori_loop(1, npairs - 1, body, 0)

            for b in range(2):
                ci = 2 * (npairs - 1) + b
                step(ci, b, True, ci + 2 < cpb)
            if cpb % 2:
                ci = cpb - 1
                step(ci, ci % 2, True, False)

            for b, (fb, ss) in enumerate(fbufs):
                pltpu.make_async_copy(fb, acc.at[dstv.at[b]], ss).wait()

        plsc.subcore_barrier()
        pltpu.sync_copy(acc.at[pl.ds(s * rps, rps), :],
                        out_hbm.at[c, pl.ds(s * rps, rps), :])

    return k


def _mm(a, b):
    return jnp.dot(a, b, preferred_element_type=jnp.float32,
                   precision=lax.Precision.HIGHEST)


def _pack_bf16(y):
    h = y.shape[1] // 2
    yh = y.astype(jnp.bfloat16).astype(jnp.float32)
    lo = lax.bitcast_convert_type(yh[:, :h], jnp.uint32)
    hi = lax.bitcast_convert_type(yh[:, h:], jnp.uint32)
    return (hi & jnp.uint32(0xFFFF0000)) | (lo >> 16)


def _unperm_np(d):
    h = d // 2
    tau = []
    for q in range(d // 32):
        tau += [16 * q + l for l in range(16)]
        tau += [16 * q + l + h for l in range(16)]
    m = np.zeros((d, d), np.float32)
    m[np.arange(d), np.array(tau)] = 1.0
    return m


def _tc1_body(x_ref, w1_ref, d0_ref, d1_ref, dinv_ref, y1_ref, pk_ref):
    dinv = lax.rsqrt(d0_ref[...] + d1_ref[...] + 1.0)
    dinv_ref[...] = dinv
    y1 = _mm(x_ref[...], w1_ref[...]) * dinv
    y1_ref[...] = y1
    pk_ref[...] = _pack_bf16(y1)


def _tc2_body(n, sp_ref, y1_ref, dinv_ref, b1_ref, w2_ref, m_ref,
              y2_ref, pk_ref):
    dinv = dinv_ref[...]
    s1 = _mm(sp_ref[0, :n, :] + sp_ref[1, :n, :], m_ref[...])
    h1 = jnp.maximum(dinv * (s1 + y1_ref[...]) + b1_ref[...], 0.0)
    y2 = _mm(h1, w2_ref[...]) * dinv
    y2_ref[...] = y2
    pk_ref[...] = _pack_bf16(y2)


def _tc3_body(n, sp_ref, y2_ref, dinv_ref, b2_ref, wlt_ref, bl_ref, m_ref,
              h2_ref, out_ref):
    dinv = dinv_ref[...]
    s2 = _mm(sp_ref[0, :n, :] + sp_ref[1, :n, :], m_ref[...])
    h2 = jnp.maximum(dinv * (s2 + y2_ref[...]) + b2_ref[...], 0.0)
    h2_ref[...] = h2
    out_ref[...] = _mm(h2, wlt_ref[...]) + bl_ref[...]


def kernel(x, edge_index, W1, b1, W2, b2, Wl, bl):
    n, d_in = x.shape
    d_hid = W1.shape[1]
    d_out = W2.shape[1]
    e = edge_index.shape[1]
    epw = e // NW
    nchunk = epw // K
    n_pad = ((n + NS * K - 1) // (NS * K)) * (NS * K)

    src3 = edge_index[0].reshape(NW, nchunk, K)
    dst3 = edge_index[1].reshape(NW, nchunk, K)
    m_hid = jnp.asarray(_unperm_np(d_hid))
    m_out = jnp.asarray(_unperm_np(d_out))

    deg_p = _make_deg_kernel(n_pad, nchunk)(dst3)
    deg0 = deg_p[0, :n].reshape(n, 1)
    deg1 = deg_p[1, :n].reshape(n, 1)

    dinv, y1, y1pk = pl.pallas_call(
        _tc1_body,
        out_shape=[jax.ShapeDtypeStruct((n, 1), jnp.float32),
                   jax.ShapeDtypeStruct((n, d_hid), jnp.float32),
                   jax.ShapeDtypeStruct((n, d_hid // 2), jnp.uint32)],
    )(x, W1, deg0, deg1)

    s1_p = _make_agg_kernel(n_pad, nchunk, d_hid, 5)(y1pk, src3, dst3)

    y2, y2pk = pl.pallas_call(
        functools.partial(_tc2_body, n),
        out_shape=[jax.ShapeDtypeStruct((n, d_out), jnp.float32),
                   jax.ShapeDtypeStruct((n, d_out // 2), jnp.uint32)],
    )(s1_p, y1, dinv, b1.reshape(1, d_hid), W2, m_hid)

    s2_p = _make_agg_kernel(n_pad, nchunk, d_out, 1)(y2pk, src3, dst3)

    h2, out = pl.pallas_call(
        functools.partial(_tc3_body, n),
        out_shape=[jax.ShapeDtypeStruct((n, d_out), jnp.float32),
                   jax.ShapeDtypeStruct((n, d_out), jnp.float32)],
    )(s2_p, y2, dinv, b2.reshape(1, d_out), Wl.T, bl.reshape(1, d_out), m_out)

    return (h2, out)


# speedup vs baseline: 1.1538x; 1.1538x over previous
import functools

import jax
import jax.numpy as jnp
from jax import lax
from jax.experimental import pallas as pl
from jax.experimental.pallas import tpu as pltpu
from jax.experimental.pallas import tpu_sc as plsc

NC = 2
NS = 16
NW = NC * NS
K = 80


def _flat_wid():
    return lax.axis_index("s") * NC + lax.axis_index("c")


def _make_deg_kernel(n_pad, nchunk):
    mesh = plsc.VectorSubcoreMesh(core_axis_name="c", subcore_axis_name="s")
    rps = n_pad // NS

    @functools.partial(
        pl.kernel,
        out_type=jax.ShapeDtypeStruct((NC, n_pad), jnp.float32),
        mesh=mesh,
        scratch_types=[
            pltpu.VMEM_SHARED((n_pad,), jnp.float32),
            pltpu.VMEM((nchunk, K), jnp.int32),
            pltpu.VMEM((K,), jnp.float32),
            pltpu.VMEM((rps,), jnp.float32),
        ],
    )
    def k(dst_hbm, out_hbm, acc, dstv, ones_v, zbuf):
        c = lax.axis_index("c")
        s = lax.axis_index("s")
        wid = _flat_wid()

        for i in range(K // 16):
            ones_v[pl.ds(16 * i, 16)] = jnp.ones((16,), jnp.float32)

        def zfill(i, _):
            zbuf[pl.ds(16 * i, 16)] = jnp.zeros((16,), jnp.float32)
            return 0
        lax.fori_loop(0, rps // 16, zfill, 0)
        pltpu.sync_copy(zbuf, acc.at[pl.ds(s * rps, rps)])

        pltpu.sync_copy(dst_hbm.at[wid], dstv)
        plsc.subcore_barrier()

        def body(ci, _):
            pltpu.sync_copy(ones_v, acc.at[dstv.at[ci]], add=True)
            return 0
        lax.fori_loop(0, nchunk, body, 0)

        plsc.subcore_barrier()
        pltpu.sync_copy(acc.at[pl.ds(s * rps, rps)],
                        out_hbm.at[c, pl.ds(s * rps, rps)])

    return k


def _make_agg_kernel(n_pad, nchunk, d, nblk):
    mesh = plsc.VectorSubcoreMesh(core_axis_name="c", subcore_axis_name="s")
    rps = n_pad // NS
    cpb = nchunk // nblk
    assert cpb >= 6

    @functools.partial(
        pl.kernel,
        out_type=jax.ShapeDtypeStruct((NC, n_pad, d), jnp.float32),
        mesh=mesh,
        scratch_types=[
            pltpu.VMEM_SHARED((n_pad, d), jnp.float32),
            pltpu.VMEM((cpb, K), jnp.int32),
            pltpu.VMEM((cpb, K), jnp.int32),
            pltpu.VMEM((K, d), jnp.float32),
            pltpu.VMEM((K, d), jnp.float32),
            pltpu.VMEM((K, d), jnp.float32),
            pltpu.SemaphoreType.DMA,
            pltpu.SemaphoreType.DMA,
            pltpu.SemaphoreType.DMA,
            pltpu.SemaphoreType.DMA,
            pltpu.SemaphoreType.DMA,
            pltpu.SemaphoreType.DMA,
        ],
        compiler_params=pltpu.CompilerParams(use_tc_tiling_on_sc=False),
    )
    def k(y_hbm, src_hbm, dst_hbm, out_hbm, acc, srcv, dstv, r0, r1, r2,
          g0, g1, g2, s0, s1, s2):
        c = lax.axis_index("c")
        s = lax.axis_index("s")
        wid = _flat_wid()
        rbuf = (r0, r1, r2)
        gsem = (g0, g1, g2)
        ssem = (s0, s1, s2)

        def zfill(i, _):
            for j in range(d // 16):
                r0[i, pl.ds(16 * j, 16)] = jnp.zeros((16,), jnp.float32)
            return 0
        lax.fori_loop(0, K, zfill, 0)
        for t in range(rps // K):
            pltpu.sync_copy(r0, acc.at[pl.ds(s * rps + t * K, K), :])
        plsc.subcore_barrier()

        def gissue(ci, b):
            pltpu.async_copy(y_hbm.at[srcv.at[ci]], rbuf[b], gsem[b])

        def gwait(ci, b):
            pltpu.make_async_copy(y_hbm.at[srcv.at[ci]], rbuf[b],
                                  gsem[b]).wait()

        def sissue(ci, b):
            pltpu.async_copy(rbuf[b], acc.at[dstv.at[ci]], ssem[b], add=True)

        def swait(ci, b):
            pltpu.make_async_copy(rbuf[b], acc.at[dstv.at[ci]],
                                  ssem[b]).wait()

        nsteady = cpb - 4
        ngrp, nrem = divmod(nsteady, 3)

        for blk in range(nblk):
            pltpu.sync_copy(src_hbm.at[wid, pl.ds(blk * cpb, cpb), :], srcv)
            pltpu.sync_copy(dst_hbm.at[wid, pl.ds(blk * cpb, cpb), :], dstv)
            gissue(0, 0)
            gissue(1, 1)
            gwait(0, 0); sissue(0, 0); gissue(2, 2)
            gwait(1, 1); sissue(1, 1); swait(0, 0); gissue(3, 0)

            def sbody(g, _):
                base = 2 + 3 * g
                for j in range(3):
                    ci = base + j
                    b = (2 + j) % 3
                    pb = (1 + j) % 3
                    gwait(ci, b); sissue(ci, b); swait(ci, pb)
                    gissue(ci + 2, pb)
                return 0
            lax.fori_loop(0, ngrp, sbody, 0)

            for j in range(nrem):
                ci = 2 + 3 * ngrp + j
                gwait(ci, (2 + j) % 3); sissue(ci, (2 + j) % 3)
                swait(ci, (1 + j) % 3); gissue(ci + 2, (1 + j) % 3)

            for ci in (cpb - 2, cpb - 1):
                gwait(ci, ci % 3); sissue(ci, ci % 3); swait(ci, (ci - 1) % 3)
            swait(0, (cpb - 1) % 3)

        plsc.subcore_barrier()
        pltpu.sync_copy(acc.at[pl.ds(s * rps, rps), :],
                        out_hbm.at[c, pl.ds(s * rps, rps), :])

    return k


def _mm(a, b):
    return jnp.dot(a, b, preferred_element_type=jnp.float32,
                   precision=lax.Precision.HIGHEST)


def _tc1_body(x_ref, w1_ref, d0_ref, d1_ref, dinv_ref, y1_ref):
    dinv = lax.rsqrt(d0_ref[...] + d1_ref[...] + 1.0)
    dinv_ref[...] = dinv
    y1_ref[...] = _mm(x_ref[...], w1_ref[...]) * dinv


def _tc2_body(n, sp_ref, y1_ref, dinv_ref, b1_ref, w2_ref, y2_ref):
    dinv = dinv_ref[...]
    s1 = sp_ref[0, :n, :] + sp_ref[1, :n, :]
    h1 = jnp.maximum(dinv * (s1 + y1_ref[...]) + b1_ref[...], 0.0)
    y2_ref[...] = _mm(h1, w2_ref[...]) * dinv


def _tc3_body(n, sp_ref, y2_ref, dinv_ref, b2_ref, wlt_ref, bl_ref,
              h2_ref, out_ref):
    dinv = dinv_ref[...]
    s2 = sp_ref[0, :n, :] + sp_ref[1, :n, :]
    h2 = jnp.maximum(dinv * (s2 + y2_ref[...]) + b2_ref[...], 0.0)
    h2_ref[...] = h2
    out_ref[...] = _mm(h2, wlt_ref[...]) + bl_ref[...]


def kernel(x, edge_index, W1, b1, W2, b2, Wl, bl):
    n, d_in = x.shape
    d_hid = W1.shape[1]
    d_out = W2.shape[1]
    e = edge_index.shape[1]
    epw = e // NW
    nchunk = epw // K
    n_pad = ((n + NS * K - 1) // (NS * K)) * (NS * K)

    src3 = edge_index[0].reshape(NW, nchunk, K)
    dst3 = edge_index[1].reshape(NW, nchunk, K)

    deg_p = _make_deg_kernel(n_pad, nchunk)(dst3)
    deg0 = deg_p[0, :n].reshape(n, 1)
    deg1 = deg_p[1, :n].reshape(n, 1)

    dinv, y1 = pl.pallas_call(
        _tc1_body,
        out_shape=[jax.ShapeDtypeStruct((n, 1), jnp.float32),
                   jax.ShapeDtypeStruct((n, d_hid), jnp.float32)],
    )(x, W1, deg0, deg1)

    s1_p = _make_agg_kernel(n_pad, nchunk, d_hid, 5)(y1, src3, dst3)

    y2 = pl.pallas_call(
        functools.partial(_tc2_body, n),
        out_shape=jax.ShapeDtypeStruct((n, d_out), jnp.float32),
    )(s1_p, y1, dinv, b1.reshape(1, d_hid), W2)

    s2_p = _make_agg_kernel(n_pad, nchunk, d_out, 1)(y2, src3, dst3)

    h2, out = pl.pallas_call(
        functools.partial(_tc3_body, n),
        out_shape=[jax.ShapeDtypeStruct((n, d_out), jnp.float32),
                   jax.ShapeDtypeStruct((n, d_out), jnp.float32)],
    )(s2_p, y2, dinv, b2.reshape(1, d_out), Wl.T, bl.reshape(1, d_out))

    return (h2, out)
